# baseline (speedup 1.0000x reference)
import functools

import jax
import jax.numpy as jnp
from jax import lax
from jax.experimental import pallas as pl
from jax.experimental.pallas import tpu as pltpu
from jax.experimental.pallas import tpu_sc as plsc

_NUM_EMBS = 1000000
_EMB_DIM = 64
_BATCH = 4096
_HIST = 50
_N = _BATCH * _HIST

_NC = 2
_NS = 16
_NW = _NC * _NS
_NPW = _N // _NW
_CHUNK = 256
_NCHUNKS = _NPW // _CHUNK
_PAIR_W = 2 * _EMB_DIM


@functools.lru_cache(maxsize=None)
def _make_gather():
    mesh = plsc.VectorSubcoreMesh(core_axis_name="c", subcore_axis_name="s")

    @functools.partial(
        pl.kernel,
        mesh=mesh,
        out_type=jax.ShapeDtypeStruct((_N * _EMB_DIM,), jnp.float32),
        compiler_params=pltpu.CompilerParams(needs_layout_passes=False),
        scratch_types=[
            pltpu.VMEM((_CHUNK,), jnp.int32),
            pltpu.VMEM((_CHUNK,), jnp.int32),
            pltpu.VMEM((_CHUNK,), jnp.int32),
            pltpu.VMEM((_CHUNK,), jnp.int32),
            pltpu.VMEM((_CHUNK,), jnp.int32),
            pltpu.VMEM((_CHUNK,), jnp.int32),
            pltpu.VMEM((_CHUNK, _PAIR_W), jnp.float32),
            pltpu.VMEM((_CHUNK, _PAIR_W), jnp.float32),
            pltpu.VMEM((_CHUNK * _EMB_DIM,), jnp.float32),
            pltpu.VMEM((_CHUNK * _EMB_DIM,), jnp.float32),
            pltpu.SemaphoreType.DMA,
            pltpu.SemaphoreType.DMA,
            pltpu.SemaphoreType.DMA,
            pltpu.SemaphoreType.DMA,
        ],
    )
    def gather(ids_hbm, z_hbm, out_hbm, idx0, idx1, pidx0, pidx1, par0,
               par1, rows0, rows1, sel0, sel1, gsem0, gsem1, osem0, osem1):
        idx_b = (idx0, idx1)
        pidx_b = (pidx0, pidx1)
        par_b = (par0, par1)
        rows_b = (rows0, rows1)
        sel_b = (sel0, sel1)
        gsem_b = (gsem0, gsem1)
        osem_b = (osem0, osem1)

        wid = lax.axis_index("s") * _NC + lax.axis_index("c")
        base = wid * _NPW

        def stage(g, b):
            off = base + g * _CHUNK
            pltpu.sync_copy(ids_hbm.at[pl.ds(off, _CHUNK)], idx_b[b])
            for i in range(_CHUNK // 16):
                v = idx_b[b][pl.ds(16 * i, 16)]
                pidx_b[b][pl.ds(16 * i, 16)] = v >> 1
                par_b[b][pl.ds(16 * i, 16)] = v & 1

        def start_gather(b):
            return pltpu.async_copy(z_hbm.at[pidx_b[b]], rows_b[b], gsem_b[b])

        def select(b):
            lanes = lax.iota(jnp.int32, 16)

            def body(t, _):
                tvec = jnp.full((16,), t, jnp.int32)
                pv = plsc.load_gather(par_b[b], [tvec])
                colb = pv * _EMB_DIM + lanes
                for k in range(_EMB_DIM // 16):
                    v = plsc.load_gather(rows_b[b], [tvec, colb + 16 * k])
                    sel_b[b][pl.ds(t * _EMB_DIM + 16 * k, 16)] = v
                return 0

            lax.fori_loop(0, _CHUNK, body, 0)

        def write_out(g, b):
            out_base = (base + g * _CHUNK) * _EMB_DIM
            return pltpu.async_copy(
                sel_b[b], out_hbm.at[pl.ds(out_base, _CHUNK * _EMB_DIM)],
                osem_b[b])

        ghandles = {}
        ohandles = {}
        stage(0, 0)
        ghandles[0] = start_gather(0)
        for g in range(1, _NCHUNKS):
            b, pb = g % 2, (g - 1) % 2
            stage(g, b)
            ghandles[b] = start_gather(b)
            ghandles[pb].wait()
            if pb in ohandles:
                ohandles[pb].wait()
            select(pb)
            ohandles[pb] = write_out(g - 1, pb)
        lb = (_NCHUNKS - 1) % 2
        ghandles[lb].wait()
        if lb in ohandles:
            ohandles[lb].wait()
        select(lb)
        ohandles[lb] = write_out(_NCHUNKS - 1, lb)
        for b in (0, 1):
            if b in ohandles:
                ohandles[b].wait()

    return gather


def kernel(padded_token_ids, table):
    ids = padded_token_ids.reshape(-1).astype(jnp.int32)
    z = table.reshape(_NUM_EMBS // 2, _PAIR_W)
    out = _make_gather()(ids, z)
    return out.reshape(_BATCH, _HIST, _EMB_DIM)


# speedup vs baseline: 1.1706x; 1.1706x over previous
import functools

import jax
import jax.numpy as jnp
from jax import lax
from jax.experimental import pallas as pl
from jax.experimental.pallas import tpu as pltpu
from jax.experimental.pallas import tpu_sc as plsc

_NUM_EMBS = 1000000
_EMB_DIM = 64
_BATCH = 4096
_HIST = 50
_N = _BATCH * _HIST

_NC = 2
_NS = 16
_NW = _NC * _NS
_NPW = _N // _NW
_CHUNK = 800
_NCHUNKS = _NPW // _CHUNK


@functools.lru_cache(maxsize=None)
def _make_gather():
    mesh = plsc.VectorSubcoreMesh(core_axis_name="c", subcore_axis_name="s")

    @functools.partial(
        pl.kernel,
        mesh=mesh,
        out_type=jax.ShapeDtypeStruct((_N, _EMB_DIM), jnp.float32),
        compiler_params=pltpu.CompilerParams(use_tc_tiling_on_sc=False),
        scratch_types=[
            pltpu.VMEM((_CHUNK,), jnp.int32),
            pltpu.VMEM((_CHUNK, _EMB_DIM), jnp.float32),
            pltpu.SemaphoreType.DMA,
        ],
    )
    def gather(ids_hbm, table_hbm, out_hbm, idx_v, rows_v, sem):
        wid = lax.axis_index("s") * _NC + lax.axis_index("c")
        base = wid * _NPW
        for g in range(_NCHUNKS):
            off = base + g * _CHUNK
            pltpu.sync_copy(ids_hbm.at[pl.ds(off, _CHUNK)], idx_v)
            pltpu.async_copy(table_hbm.at[idx_v], rows_v, sem).wait()
            pltpu.sync_copy(rows_v, out_hbm.at[pl.ds(off, _CHUNK), :])

    return gather


def kernel(padded_token_ids, table):
    ids = padded_token_ids.reshape(-1).astype(jnp.int32)
    out = _make_gather()(ids, table)
    return out.reshape(_BATCH, _HIST, _EMB_DIM)
